# baseline (speedup 1.0000x reference)
import functools

import jax
import jax.numpy as jnp
from jax import lax
from jax.experimental import pallas as pl
from jax.experimental.pallas import tpu as pltpu
from jax.experimental.pallas import tpu_sc as plsc

N = 10000
E = 320000
NP = 10112
NW = 32
EPW = 10240
EP = NW * EPW
CHUNK = 128
NCHUNK = EPW // CHUNK
ROWS_PER_SUB = NP // 16

_mesh = plsc.VectorSubcoreMesh(core_axis_name="c", subcore_axis_name="s",
                               num_cores=2, num_subcores=16)
_sc_params = pltpu.CompilerParams(needs_layout_passes=False,
                                 use_tc_tiling_on_sc=False)


def _wid():
    return lax.axis_index("s") * 2 + lax.axis_index("c")


@functools.partial(
    pl.kernel,
    out_type=jax.ShapeDtypeStruct((NW * NP,), jnp.float32),
    mesh=_mesh,
    compiler_params=_sc_params,
    scratch_types=[
        pltpu.VMEM((EPW,), jnp.int32),
        pltpu.VMEM((NP,), jnp.float32),
    ],
)
def _sc_degree(dst_hbm, out_hbm, dst_v, acc_v):
    w = _wid()
    pltpu.sync_copy(dst_hbm.at[pl.ds(w * EPW, EPW)], dst_v)

    @pl.loop(0, NP, step=16)
    def _zero(i):
        acc_v[pl.ds(i, 16)] = jnp.zeros((16,), jnp.float32)

    ones = jnp.ones((16,), jnp.float32)

    @pl.loop(0, EPW, step=16)
    def _count(k):
        idx = dst_v[pl.ds(k, 16)]
        plsc.addupdate_scatter(acc_v, [idx], ones)

    pltpu.sync_copy(acc_v, out_hbm.at[pl.ds(w * NP, NP)])


_NBUF = 4
_C0 = 40
_C1 = 2 * (NCHUNK) - _C0
_CMAX = max(_C0, _C1)
_NCH_TOT = 16 * (_C0 + _C1)
_SLAB_ROWS = 16 * _C0 + 16 * _C1 + abs(_C0 - _C1)


@functools.partial(
    pl.kernel,
    out_type=jax.ShapeDtypeStruct((2, NP, 64), jnp.float32),
    mesh=_mesh,
    compiler_params=_sc_params,
    scratch_types=[
        pltpu.VMEM((_CMAX, CHUNK), jnp.int32),
        pltpu.VMEM((_CMAX, CHUNK), jnp.int32),
        pltpu.VMEM((_NBUF * CHUNK, 64), jnp.float32),
        pltpu.VMEM_SHARED((NP, 64), jnp.float32),
    ] + [pltpu.SemaphoreType.DMA] * (2 * _NBUF),
)
def _sc_edge_rows(hs_hbm, src_hbm, dst_hbm, out_hbm, src_v, dst_v, vals_v,
                  acc_sh, *sems):
    gsem = sems[:_NBUF]
    ssem = sems[_NBUF:]
    c = lax.axis_index("c")
    s = lax.axis_index("s")
    cnt = jnp.where(c == 0, _C0, _C1)
    base = jnp.where(c == 0, s * _C0, 16 * _C0 + s * _C1)
    pltpu.sync_copy(src_hbm.at[pl.ds(base, _CMAX)], src_v)
    pltpu.sync_copy(dst_hbm.at[pl.ds(base, _CMAX)], dst_v)
    row0 = s * ROWS_PER_SUB
    pltpu.sync_copy(hs_hbm.at[pl.ds(row0, ROWS_PER_SUB)],
                    acc_sh.at[pl.ds(row0, ROWS_PER_SUB)])
    plsc.subcore_barrier()

    bufs = [vals_v.at[pl.ds(b * CHUNK, CHUNK)] for b in range(_NBUF)]
    for b in range(_NBUF):
        pltpu.async_copy(hs_hbm.at[src_v.at[b]], bufs[b], gsem[b])

    @pl.loop(0, _CMAX, step=_NBUF)
    def _edges(j0):
        for b in range(_NBUF):
            ch = j0 + b

            @pl.when(ch < cnt)
            def _fire():
                pltpu.make_async_copy(hs_hbm.at[src_v.at[ch]], bufs[b],
                                      gsem[b]).wait()
                pltpu.async_copy(bufs[b], acc_sh.at[dst_v.at[ch]],
                                 ssem[b], add=True)
        for b in range(_NBUF):
            ch = j0 + b
            nxt = ch + _NBUF

            @pl.when(ch < cnt)
            def _drain():
                pltpu.make_async_copy(bufs[b], acc_sh.at[dst_v.at[ch]],
                                      ssem[b]).wait()

                @pl.when(nxt < cnt)
                def _refill():
                    pltpu.async_copy(hs_hbm.at[src_v.at[nxt]], bufs[b],
                                     gsem[b])

    plsc.subcore_barrier()
    pltpu.sync_copy(acc_sh.at[pl.ds(row0, ROWS_PER_SUB)],
                    out_hbm.at[c].at[pl.ds(row0, ROWS_PER_SUB)])


@functools.partial(
    pl.kernel,
    out_type=jax.ShapeDtypeStruct((NW * NP,), jnp.float32),
    mesh=_mesh,
    compiler_params=_sc_params,
    scratch_types=[
        pltpu.VMEM((EPW,), jnp.int32),
        pltpu.VMEM((EPW,), jnp.int32),
        pltpu.VMEM((NP,), jnp.float32),
        pltpu.VMEM((NP,), jnp.float32),
    ],
)
def _sc_edge_scalar(us_hbm, src_hbm, dst_hbm, out_hbm, src_v, dst_v, us_v,
                    acc_v):
    w = _wid()
    pltpu.sync_copy(src_hbm.at[pl.ds(w * EPW, EPW)], src_v)
    pltpu.sync_copy(dst_hbm.at[pl.ds(w * EPW, EPW)], dst_v)
    pltpu.sync_copy(us_hbm, us_v)

    @pl.loop(0, NP, step=16)
    def _zero(i):
        acc_v[pl.ds(i, 16)] = jnp.zeros((16,), jnp.float32)

    @pl.loop(0, EPW, step=16)
    def _edges(k):
        si = src_v[pl.ds(k, 16)]
        di = dst_v[pl.ds(k, 16)]
        vals = plsc.load_gather(us_v, [si])
        plsc.addupdate_scatter(acc_v, [di], vals)

    pltpu.sync_copy(acc_v, out_hbm.at[pl.ds(w * NP, NP)])


_RB = 1000


def _prescale_body(x_ref, w1_ref, degt_ref, hs_ref, dinv_ref):
    deg = jnp.sum(degt_ref[...], axis=1, keepdims=True) + 1.0
    dinv = lax.rsqrt(deg)
    h = jnp.dot(x_ref[...], w1_ref[...], preferred_element_type=jnp.float32)
    hs_ref[...] = h * dinv
    dinv_ref[...] = dinv


def _tc_prescale(x, W1, degT):
    return pl.pallas_call(
        _prescale_body,
        grid=(N // _RB,),
        in_specs=[
            pl.BlockSpec((_RB, 128), lambda i: (i, 0)),
            pl.BlockSpec((128, 64), lambda i: (0, 0)),
            pl.BlockSpec((_RB, NW), lambda i: (i, 0)),
        ],
        out_specs=[
            pl.BlockSpec((_RB, 64), lambda i: (i, 0)),
            pl.BlockSpec((_RB, 1), lambda i: (i, 0)),
        ],
        out_shape=[
            jax.ShapeDtypeStruct((N, 64), jnp.float32),
            jax.ShapeDtypeStruct((N, 1), jnp.float32),
        ],
    )(x, W1, degT)


def _mid_body(acc_ref, hs_ref, dinv_ref, b1_ref, w2_ref, us_ref):
    agg = (acc_ref[0] + acc_ref[1] - hs_ref[...]) * dinv_ref[...]
    h1 = jnp.maximum(agg + b1_ref[...], 0.0)
    h1q = h1.astype(jnp.bfloat16).astype(jnp.float32)
    w2q = w2_ref[...].astype(jnp.bfloat16).astype(jnp.float32)
    u = jnp.sum(h1q * w2q, axis=1, keepdims=True)
    us_ref[...] = u * dinv_ref[...]


def _tc_mid(acc, hs, dinv_col, b1_row, w2_row):
    return pl.pallas_call(
        _mid_body,
        grid=(N // _RB,),
        in_specs=[
            pl.BlockSpec((2, _RB, 64), lambda i: (0, i, 0)),
            pl.BlockSpec((_RB, 64), lambda i: (i, 0)),
            pl.BlockSpec((_RB, 1), lambda i: (i, 0)),
            pl.BlockSpec((1, 64), lambda i: (0, 0)),
            pl.BlockSpec((1, 64), lambda i: (0, 0)),
        ],
        out_specs=pl.BlockSpec((_RB, 1), lambda i: (i, 0)),
        out_shape=jax.ShapeDtypeStruct((N, 1), jnp.float32),
    )(acc, hs, dinv_col, b1_row, w2_row)


_IB = 1000
_JB = 2048
_NJ = -(-N // _JB)


def _head1_body(us_ref, acct_ref, dinv_ref, b2_ref, w_ref, bias_ref, o_ref):
    i = pl.program_id(1)
    flat = (us_ref[...] + jnp.sum(acct_ref[...], axis=1, keepdims=True))
    flat = flat * dinv_ref[...] + b2_ref[...]

    @pl.when(i == 0)
    def _():
        o_ref[...] = jnp.zeros_like(o_ref)

    flatq = flat.astype(jnp.bfloat16).astype(jnp.float32)
    wq = w_ref[...].astype(jnp.bfloat16).astype(jnp.float32)
    o_ref[...] += jnp.sum(wq * flatq, axis=0, keepdims=True)

    @pl.when(i == (N // _IB) - 1)
    def _():
        o_ref[...] = jnp.maximum(o_ref[...] + bias_ref[...], 0.0)


def _tc_head1(us_col, acc1T, dinv_col, b2_11, Wl1, bl1_row):
    return pl.pallas_call(
        _head1_body,
        grid=(_NJ, N // _IB),
        in_specs=[
            pl.BlockSpec((_IB, 1), lambda j, i: (i, 0)),
            pl.BlockSpec((_IB, NW), lambda j, i: (i, 0)),
            pl.BlockSpec((_IB, 1), lambda j, i: (i, 0)),
            pl.BlockSpec((1, 1), lambda j, i: (0, 0)),
            pl.BlockSpec((_IB, _JB), lambda j, i: (i, j)),
            pl.BlockSpec((1, _JB), lambda j, i: (0, j)),
        ],
        out_specs=pl.BlockSpec((1, _JB), lambda j, i: (0, j)),
        out_shape=jax.ShapeDtypeStruct((1, N), jnp.float32),
    )(us_col, acc1T, dinv_col, b2_11, Wl1, bl1_row)


def _head2_body(o_ref, w_ref, bias_ref, out_ref):
    i = pl.program_id(1)

    @pl.when(i == 0)
    def _():
        out_ref[...] = jnp.zeros_like(out_ref)

    oq = o_ref[...].astype(jnp.bfloat16).astype(jnp.float32)
    wq = w_ref[...].astype(jnp.bfloat16).astype(jnp.float32)
    out_ref[...] += jnp.sum(wq * oq, axis=0, keepdims=True)

    @pl.when(i == (N // _IB) - 1)
    def _():
        out_ref[...] += bias_ref[...]


def _tc_head2(o_col, Wl2, bl2_row):
    return pl.pallas_call(
        _head2_body,
        grid=(_NJ, N // _IB),
        in_specs=[
            pl.BlockSpec((_IB, 1), lambda j, i: (i, 0)),
            pl.BlockSpec((_IB, _JB), lambda j, i: (i, j)),
            pl.BlockSpec((1, _JB), lambda j, i: (0, j)),
        ],
        out_specs=pl.BlockSpec((1, _JB), lambda j, i: (0, j)),
        out_shape=jax.ShapeDtypeStruct((1, N), jnp.float32),
    )(o_col, Wl2, bl2_row)


@jax.jit
def kernel(x, edge_index, W1, b1, W2, b2, Wl1, bl1, Wl2, bl2):
    src = edge_index[0].astype(jnp.int32)
    dst = edge_index[1].astype(jnp.int32)
    pad = jnp.full((EP - E,), N, jnp.int32)
    src_flat = jnp.concatenate([src, pad])
    dst_flat = jnp.concatenate([dst, pad])
    tailpad = jnp.full((abs(_C0 - _C1) * CHUNK,), N, jnp.int32)
    src2 = jnp.concatenate([src_flat, tailpad]).reshape(_SLAB_ROWS, CHUNK)
    dst2 = jnp.concatenate([dst_flat, tailpad]).reshape(_SLAB_ROWS, CHUNK)

    deg_part = _sc_degree(dst_flat).reshape(NW, NP)
    degT = deg_part[:, :N].T

    hs, dinv_col = _tc_prescale(x, W1, degT)
    hs_p = jnp.pad(hs, ((0, NP - N), (0, 0)))

    acc = _sc_edge_rows(hs_p, src2, dst2)
    acc = acc[:, :N, :]

    us_col = _tc_mid(acc, hs, dinv_col, b1.reshape(1, 64),
                     W2.reshape(1, 64))
    us_p = jnp.pad(us_col[:, 0], (0, NP - N))

    acc1 = _sc_edge_scalar(us_p, src_flat,
                           dst_flat).reshape(NW, NP)
    acc1T = acc1[:, :N].T

    o_row = _tc_head1(us_col, acc1T, dinv_col, b2.reshape(1, 1),
                      Wl1, bl1.reshape(1, N))
    logits = _tc_head2(o_row.T, Wl2, bl2.reshape(1, N))
    return logits


# speedup vs baseline: 1.0921x; 1.0921x over previous
import functools

import jax
import jax.numpy as jnp
from jax import lax
from jax.experimental import pallas as pl
from jax.experimental.pallas import tpu as pltpu
from jax.experimental.pallas import tpu_sc as plsc

N = 10000
E = 320000
NP = 10112
NW = 32
EPW = 10240
EP = NW * EPW
CHUNK = 128
NCHUNK = EPW // CHUNK
ROWS_PER_SUB = NP // 16

_mesh = plsc.VectorSubcoreMesh(core_axis_name="c", subcore_axis_name="s",
                               num_cores=2, num_subcores=16)
_sc_params = pltpu.CompilerParams(needs_layout_passes=False,
                                 use_tc_tiling_on_sc=False)


def _wid():
    return lax.axis_index("s") * 2 + lax.axis_index("c")


@functools.partial(
    pl.kernel,
    out_type=jax.ShapeDtypeStruct((NW * NP,), jnp.float32),
    mesh=_mesh,
    compiler_params=_sc_params,
    scratch_types=[
        pltpu.VMEM((EPW,), jnp.int32),
        pltpu.VMEM((NP,), jnp.float32),
    ],
)
def _sc_degree(dst_hbm, out_hbm, dst_v, acc_v):
    w = _wid()
    pltpu.sync_copy(dst_hbm.at[pl.ds(w * EPW, EPW)], dst_v)

    @pl.loop(0, NP, step=16)
    def _zero(i):
        acc_v[pl.ds(i, 16)] = jnp.zeros((16,), jnp.float32)

    ones = jnp.ones((16,), jnp.float32)

    @pl.loop(0, EPW, step=16)
    def _count(k):
        idx = dst_v[pl.ds(k, 16)]
        plsc.addupdate_scatter(acc_v, [idx], ones)

    pltpu.sync_copy(acc_v, out_hbm.at[pl.ds(w * NP, NP)])


_NBUF = 4


@functools.partial(
    pl.kernel,
    out_type=jax.ShapeDtypeStruct((2, NP, 64), jnp.float32),
    mesh=_mesh,
    compiler_params=_sc_params,
    scratch_types=[
        pltpu.VMEM((NCHUNK, CHUNK), jnp.int32),
        pltpu.VMEM((NCHUNK, CHUNK), jnp.int32),
        pltpu.VMEM((_NBUF * CHUNK, 64), jnp.float32),
        pltpu.VMEM_SHARED((NP, 64), jnp.float32),
    ] + [pltpu.SemaphoreType.DMA] * (2 * _NBUF),
)
def _sc_edge_rows(hs_hbm, src_hbm, dst_hbm, out_hbm, src_v, dst_v, vals_v,
                  acc_sh, *sems):
    gsem = sems[:_NBUF]
    ssem = sems[_NBUF:]
    c = lax.axis_index("c")
    s = lax.axis_index("s")
    w = s * 2 + c
    pltpu.sync_copy(src_hbm.at[pl.ds(w * NCHUNK, NCHUNK)], src_v)
    pltpu.sync_copy(dst_hbm.at[pl.ds(w * NCHUNK, NCHUNK)], dst_v)
    row0 = s * ROWS_PER_SUB
    pltpu.sync_copy(hs_hbm.at[pl.ds(row0, ROWS_PER_SUB)],
                    acc_sh.at[pl.ds(row0, ROWS_PER_SUB)])
    plsc.subcore_barrier()

    bufs = [vals_v.at[pl.ds(b * CHUNK, CHUNK)] for b in range(_NBUF)]
    for b in range(_NBUF):
        pltpu.async_copy(hs_hbm.at[src_v.at[b]], bufs[b], gsem[b])

    @pl.loop(0, NCHUNK, step=_NBUF)
    def _edges(j0):
        for b in range(_NBUF):
            pltpu.make_async_copy(hs_hbm.at[src_v.at[j0 + b]], bufs[b],
                                  gsem[b]).wait()
            pltpu.async_copy(bufs[b], acc_sh.at[dst_v.at[j0 + b]],
                             ssem[b], add=True)
        for b in range(_NBUF):
            pltpu.make_async_copy(bufs[b], acc_sh.at[dst_v.at[j0 + b]],
                                  ssem[b]).wait()
            nxt = j0 + _NBUF + b

            @pl.when(nxt < NCHUNK)
            def _refill():
                pltpu.async_copy(hs_hbm.at[src_v.at[nxt]], bufs[b],
                                 gsem[b])

    plsc.subcore_barrier()
    pltpu.sync_copy(acc_sh.at[pl.ds(row0, ROWS_PER_SUB)],
                    out_hbm.at[c].at[pl.ds(row0, ROWS_PER_SUB)])


@functools.partial(
    pl.kernel,
    out_type=jax.ShapeDtypeStruct((NW * NP,), jnp.float32),
    mesh=_mesh,
    compiler_params=_sc_params,
    scratch_types=[
        pltpu.VMEM((EPW,), jnp.int32),
        pltpu.VMEM((EPW,), jnp.int32),
        pltpu.VMEM((NP,), jnp.float32),
        pltpu.VMEM((NP,), jnp.float32),
    ],
)
def _sc_edge_scalar(us_hbm, src_hbm, dst_hbm, out_hbm, src_v, dst_v, us_v,
                    acc_v):
    w = _wid()
    pltpu.sync_copy(src_hbm.at[pl.ds(w * EPW, EPW)], src_v)
    pltpu.sync_copy(dst_hbm.at[pl.ds(w * EPW, EPW)], dst_v)
    pltpu.sync_copy(us_hbm, us_v)

    @pl.loop(0, NP, step=16)
    def _zero(i):
        acc_v[pl.ds(i, 16)] = jnp.zeros((16,), jnp.float32)

    @pl.loop(0, EPW, step=16)
    def _edges(k):
        si = src_v[pl.ds(k, 16)]
        di = dst_v[pl.ds(k, 16)]
        vals = plsc.load_gather(us_v, [si])
        plsc.addupdate_scatter(acc_v, [di], vals)

    pltpu.sync_copy(acc_v, out_hbm.at[pl.ds(w * NP, NP)])


_RB = 1000


def _prescale_body(x_ref, w1_ref, degt_ref, hs_ref, dinv_ref):
    deg = jnp.sum(degt_ref[...], axis=1, keepdims=True) + 1.0
    dinv = lax.rsqrt(deg)
    h = jnp.dot(x_ref[...], w1_ref[...], preferred_element_type=jnp.float32)
    hs_ref[...] = h * dinv
    dinv_ref[...] = dinv


def _tc_prescale(x, W1, degT):
    return pl.pallas_call(
        _prescale_body,
        grid=(N // _RB,),
        in_specs=[
            pl.BlockSpec((_RB, 128), lambda i: (i, 0)),
            pl.BlockSpec((128, 64), lambda i: (0, 0)),
            pl.BlockSpec((_RB, NW), lambda i: (i, 0)),
        ],
        out_specs=[
            pl.BlockSpec((_RB, 64), lambda i: (i, 0)),
            pl.BlockSpec((_RB, 1), lambda i: (i, 0)),
        ],
        out_shape=[
            jax.ShapeDtypeStruct((N, 64), jnp.float32),
            jax.ShapeDtypeStruct((N, 1), jnp.float32),
        ],
    )(x, W1, degT)


def _mid_body(acc_ref, hs_ref, dinv_ref, b1_ref, w2_ref, us_ref):
    agg = (acc_ref[0] + acc_ref[1] - hs_ref[...]) * dinv_ref[...]
    h1 = jnp.maximum(agg + b1_ref[...], 0.0)
    h1q = h1.astype(jnp.bfloat16).astype(jnp.float32)
    w2q = w2_ref[...].astype(jnp.bfloat16).astype(jnp.float32)
    u = jnp.sum(h1q * w2q, axis=1, keepdims=True)
    us_ref[...] = u * dinv_ref[...]


def _tc_mid(acc, hs, dinv_col, b1_row, w2_row):
    return pl.pallas_call(
        _mid_body,
        grid=(N // _RB,),
        in_specs=[
            pl.BlockSpec((2, _RB, 64), lambda i: (0, i, 0)),
            pl.BlockSpec((_RB, 64), lambda i: (i, 0)),
            pl.BlockSpec((_RB, 1), lambda i: (i, 0)),
            pl.BlockSpec((1, 64), lambda i: (0, 0)),
            pl.BlockSpec((1, 64), lambda i: (0, 0)),
        ],
        out_specs=pl.BlockSpec((_RB, 1), lambda i: (i, 0)),
        out_shape=jax.ShapeDtypeStruct((N, 1), jnp.float32),
    )(acc, hs, dinv_col, b1_row, w2_row)


_IB = 1000
_JB = 2048
_NJ = -(-N // _JB)


def _head1_body(us_ref, acct_ref, dinv_ref, b2_ref, w_ref, bias_ref, o_ref):
    i = pl.program_id(1)
    flat = (us_ref[...] + jnp.sum(acct_ref[...], axis=1, keepdims=True))
    flat = flat * dinv_ref[...] + b2_ref[...]

    @pl.when(i == 0)
    def _():
        o_ref[...] = jnp.zeros_like(o_ref)

    flatq = flat.astype(jnp.bfloat16).astype(jnp.float32)
    wq = w_ref[...].astype(jnp.bfloat16).astype(jnp.float32)
    o_ref[...] += jnp.sum(wq * flatq, axis=0, keepdims=True)

    @pl.when(i == (N // _IB) - 1)
    def _():
        o_ref[...] = jnp.maximum(o_ref[...] + bias_ref[...], 0.0)


def _tc_head1(us_col, acc1T, dinv_col, b2_11, Wl1, bl1_row):
    return pl.pallas_call(
        _head1_body,
        grid=(_NJ, N // _IB),
        in_specs=[
            pl.BlockSpec((_IB, 1), lambda j, i: (i, 0)),
            pl.BlockSpec((_IB, NW), lambda j, i: (i, 0)),
            pl.BlockSpec((_IB, 1), lambda j, i: (i, 0)),
            pl.BlockSpec((1, 1), lambda j, i: (0, 0)),
            pl.BlockSpec((_IB, _JB), lambda j, i: (i, j)),
            pl.BlockSpec((1, _JB), lambda j, i: (0, j)),
        ],
        out_specs=pl.BlockSpec((1, _JB), lambda j, i: (0, j)),
        out_shape=jax.ShapeDtypeStruct((1, N), jnp.float32),
    )(us_col, acc1T, dinv_col, b2_11, Wl1, bl1_row)


def _head2_body(o_ref, w_ref, bias_ref, out_ref):
    i = pl.program_id(1)

    @pl.when(i == 0)
    def _():
        out_ref[...] = jnp.zeros_like(out_ref)

    oq = o_ref[...].astype(jnp.bfloat16).astype(jnp.float32)
    wq = w_ref[...].astype(jnp.bfloat16).astype(jnp.float32)
    out_ref[...] += jnp.sum(wq * oq, axis=0, keepdims=True)

    @pl.when(i == (N // _IB) - 1)
    def _():
        out_ref[...] += bias_ref[...]


def _tc_head2(o_col, Wl2, bl2_row):
    return pl.pallas_call(
        _head2_body,
        grid=(_NJ, N // _IB),
        in_specs=[
            pl.BlockSpec((_IB, 1), lambda j, i: (i, 0)),
            pl.BlockSpec((_IB, _JB), lambda j, i: (i, j)),
            pl.BlockSpec((1, _JB), lambda j, i: (0, j)),
        ],
        out_specs=pl.BlockSpec((1, _JB), lambda j, i: (0, j)),
        out_shape=jax.ShapeDtypeStruct((1, N), jnp.float32),
    )(o_col, Wl2, bl2_row)


@jax.jit
def kernel(x, edge_index, W1, b1, W2, b2, Wl1, bl1, Wl2, bl2):
    src = edge_index[0].astype(jnp.int32)
    dst = edge_index[1].astype(jnp.int32)
    pad = jnp.full((EP - E,), N, jnp.int32)
    src_flat = jnp.concatenate([src, pad])
    dst_flat = jnp.concatenate([dst, pad])
    src2 = src_flat.reshape(NW * NCHUNK, CHUNK)
    dst2 = dst_flat.reshape(NW * NCHUNK, CHUNK)

    deg_part = _sc_degree(dst_flat).reshape(NW, NP)
    degT = deg_part[:, :N].T

    hs, dinv_col = _tc_prescale(x, W1, degT)
    hs_p = jnp.pad(hs, ((0, NP - N), (0, 0)))

    acc = _sc_edge_rows(hs_p, src2, dst2)
    acc = acc[:, :N, :]

    us_col = _tc_mid(acc, hs, dinv_col, b1.reshape(1, 64),
                     W2.reshape(1, 64))
    us_p = jnp.pad(us_col[:, 0], (0, NP - N))

    acc1 = _sc_edge_scalar(us_p, src_flat,
                           dst_flat).reshape(NW, NP)
    acc1T = acc1[:, :N].T

    o_row = _tc_head1(us_col, acc1T, dinv_col, b2.reshape(1, 1),
                      Wl1, bl1.reshape(1, N))
    logits = _tc_head2(o_row.T, Wl2, bl2.reshape(1, N))
    return logits
